# Initial kernel scaffold; baseline (speedup 1.0000x reference)
import jax
import jax.numpy as jnp
from jax.experimental import pallas as pl


def kernel(A, params):
    raise NotImplementedError("write your pallas kernel here")


# speedup vs baseline: 2.5697x; 2.5697x over previous
import jax
import jax.numpy as jnp
from jax import lax
from jax.experimental import pallas as pl
from jax.experimental.pallas import tpu as pltpu

_BN_EPS = 1e-5


def _prep_kernel(at_ref, x_ref):
    re = at_ref[0]
    im = at_ref[1]
    stre = jnp.sqrt(re * re + im * im + 1e-12)
    nrm = jnp.sqrt(jnp.sum(stre * stre, axis=1, keepdims=True) + 1e-12)
    x_ref[0] = re
    x_ref[1] = im
    x_ref[2] = stre
    x_ref[3] = stre / nrm


def _y0_block(x, sumb_blk, w0f, w0s, b0, cb, nr, nt):
    d = w0f.shape[-1]
    xr = x.reshape(4, cb, nr, nt)
    m_nr = jnp.mean(xr, axis=2)
    m_nt = jnp.mean(xr, axis=3)
    m_all = jnp.mean(m_nr, axis=2)
    cd = (((0,), (0,)), ((), ()))
    y = lax.dot_general(x.reshape(4, cb * nr * nt), w0f[0], cd,
                        preferred_element_type=jnp.float32)
    y = y.reshape(cb, nr, nt, d)
    y = y + lax.dot_general(m_nr.reshape(4, cb * nt), w0f[1], cd,
                            preferred_element_type=jnp.float32
                            ).reshape(cb, nt, d)[:, None]
    y = y + lax.dot_general(m_nt.reshape(4, cb * nr), w0f[2], cd,
                            preferred_element_type=jnp.float32
                            ).reshape(cb, nr, d)[:, :, None, :]
    y = y + lax.dot_general(m_all, w0f[3], cd,
                            preferred_element_type=jnp.float32
                            )[:, None, None, :]
    y = y + sumb_blk[:, None, None, None] * w0s[None, None, None, :]
    y = y + b0[None, None, None, :]
    return y


def _layer1_block(h, w14, b1, cb, nr, nt):
    d = w14.shape[-1]
    m_nr = jnp.mean(h, axis=1)
    m_nt = jnp.mean(h, axis=2)
    m_all = jnp.mean(m_nr, axis=1)
    y = jnp.dot(h.reshape(cb * nr * nt, d), w14[0],
                preferred_element_type=jnp.float32).reshape(cb, nr, nt, d)
    y = y + jnp.dot(m_nr.reshape(cb * nt, d), w14[1],
                    preferred_element_type=jnp.float32
                    ).reshape(cb, nt, d)[:, None]
    y = y + jnp.dot(m_nt.reshape(cb * nr, d), w14[2],
                    preferred_element_type=jnp.float32
                    ).reshape(cb, nr, d)[:, :, None, :]
    y = y + jnp.dot(m_all, w14[3],
                    preferred_element_type=jnp.float32)[:, None, None, :]
    y = y + b1[None, None, None, :]
    return y


def _bn_relu(y, st, npos):
    mu = st[0] / npos
    var = st[1] / npos - mu * mu
    rs = lax.rsqrt(var + _BN_EPS)
    return jnp.maximum((y - mu) * rs, 0.0)


def _accum_stats(st_ref, y, s):
    ps = jnp.stack([jnp.sum(y, axis=(0, 1, 2)),
                    jnp.sum(y * y, axis=(0, 1, 2))])

    @pl.when(s == 0)
    def _():
        st_ref[...] = ps

    @pl.when(s != 0)
    def _():
        st_ref[...] = st_ref[...] + ps


def _make_ph0(cb, nr, nt, k):
    def body(x_ref, sumb_ref, w0f_ref, w0s_ref, b0_ref, st_ref):
        s = pl.program_id(0)
        cbr = cb // k
        sumb_blk = sumb_ref[pl.ds(s * cbr, cbr), :].reshape(cb)
        w0s = jnp.sum(w0s_ref[...], axis=0)
        y = _y0_block(x_ref[...], sumb_blk, w0f_ref[...], w0s, b0_ref[0],
                      cb, nr, nt)
        _accum_stats(st_ref, y, s)
    return body


def _make_ph1(cb, nr, nt, k, npos):
    def body(x_ref, sumb_ref, st1_ref, w0f_ref, w0s_ref, b0_ref,
             w1_ref, b1_ref, st_ref):
        s = pl.program_id(0)
        cbr = cb // k
        sumb_blk = sumb_ref[pl.ds(s * cbr, cbr), :].reshape(cb)
        w0s = jnp.sum(w0s_ref[...], axis=0)
        y0 = _y0_block(x_ref[...], sumb_blk, w0f_ref[...], w0s, b0_ref[0],
                       cb, nr, nt)
        h0 = _bn_relu(y0, st1_ref[...], npos)
        y1 = _layer1_block(h0, w1_ref[...], b1_ref[0], cb, nr, nt)
        _accum_stats(st_ref, y1, s)
    return body


def _make_ph2(cb, nr, nt, k, npos, nsteps):
    def body(x_ref, sumb_ref, st1_ref, st2_ref, w0f_ref, w0s_ref, b0_ref,
             w1_ref, b1_ref, w2_ref, b2_ref, bout_ref, sumbn_ref,
             logit_ref):
        s = pl.program_id(0)
        cbr = cb // k
        sumb_blk = sumb_ref[pl.ds(s * cbr, cbr), :].reshape(cb)
        w0s = jnp.sum(w0s_ref[...], axis=0)
        y0 = _y0_block(x_ref[...], sumb_blk, w0f_ref[...], w0s, b0_ref[0],
                       cb, nr, nt)
        h0 = _bn_relu(y0, st1_ref[...], npos)
        y1 = _layer1_block(h0, w1_ref[...], b1_ref[0], cb, nr, nt)
        h1 = _bn_relu(y1, st2_ref[...], npos)
        mbar = jnp.mean(h1, axis=(1, 2))
        w2s = jnp.sum(w2_ref[...], axis=0)
        ylog = jnp.dot(mbar, w2s[:, None],
                       preferred_element_type=jnp.float32)
        ylog = ylog + b2_ref[0, 0]
        logit_ref[pl.ds(s * cbr, cbr), :] = ylog.reshape(cbr, k)

        @pl.when(s == nsteps - 1)
        def _():
            logits = logit_ref[...]
            mx = jnp.max(logits, axis=1, keepdims=True)
            e = jnp.exp(logits - mx)
            p = e / jnp.sum(e, axis=1, keepdims=True)
            bout_ref[...] = p
            sumbn_ref[...] = sumb_ref[...] + p
    return body


def _make_head(m, k, p):
    def body(a_ref, bb_ref, hnew_ref, x_ref, z_ref):
        ablk = a_ref[0]
        bblk = bb_ref[0]
        hn = lax.dot_general(bblk, ablk, (((2,), (2,)), ((0,), (0,))),
                             preferred_element_type=jnp.float32)
        g = jnp.sum(hn * hn, axis=(2, 3))
        hnt = jnp.transpose(hn, (0, 2, 1, 3))
        hnew_ref[0] = hnt
        x_ref[0] = hnt / (jnp.sqrt(g)[:, None, :, None] + 1e-8)
        z_ref[pl.ds(pl.program_id(0), 1), :] = (
            jnp.sum(jnp.log2(1.0 + g)).reshape(1, 1))
    return body


def kernel(A, params):
    B, C, M, K, NR, NT = A.shape
    NRF = params["W0"].shape[0]
    P = NR * NT
    ncell = B * M * K
    nbm = B * M
    npos = float(ncell * P)
    cb = min(64, ncell)
    nsteps = ncell // cb
    f32 = jnp.float32

    at = jnp.transpose(A, (1, 0, 2, 3, 4, 5)).reshape(C, ncell, P)
    pcb = min(128, ncell)
    xaug = pl.pallas_call(
        _prep_kernel,
        grid=(ncell // pcb,),
        in_specs=[pl.BlockSpec((C, pcb, P), lambda s: (0, s, 0))],
        out_specs=pl.BlockSpec((4, pcb, P), lambda s: (0, s, 0)),
        out_shape=jax.ShapeDtypeStruct((4, ncell, P), f32),
    )(at)

    full2 = lambda shape: pl.BlockSpec(shape, lambda s: tuple(0 for _ in shape))
    x_spec = pl.BlockSpec((4, cb, P), lambda s: (0, s, 0))
    st_shape = jax.ShapeDtypeStruct((2, 64), f32)

    ph0 = _make_ph0(cb, NR, NT, K)
    ph1 = _make_ph1(cb, NR, NT, K, npos)
    ph2 = _make_ph2(cb, NR, NT, K, npos, nsteps)

    W0, W1, W2 = params["W0"], params["W1"], params["W2"]
    b0a, b1a, b2a = params["b0"], params["b1"], params["b2"]

    sumb = jnp.zeros((nbm, K), f32)
    bs = []
    for i in range(NRF):
        w0f = W0[i, :, :4, :]
        w0s4 = W0[i, :, 4, :]
        b0i = b0a[i:i + 1]
        w1i = W1[i]
        b1i = b1a[i:i + 1]
        w2i = W2[i, :, :, 0]
        b2i = b2a[i:i + 1]

        st1 = pl.pallas_call(
            ph0,
            grid=(nsteps,),
            in_specs=[x_spec, full2((nbm, K)), full2((4, 4, 64)),
                      full2((4, 64)), full2((1, 64))],
            out_specs=full2((2, 64)),
            out_shape=st_shape,
        )(xaug, sumb, w0f, w0s4, b0i)

        st2 = pl.pallas_call(
            ph1,
            grid=(nsteps,),
            in_specs=[x_spec, full2((nbm, K)), full2((2, 64)),
                      full2((4, 4, 64)), full2((4, 64)), full2((1, 64)),
                      full2((4, 64, 64)), full2((1, 64))],
            out_specs=full2((2, 64)),
            out_shape=st_shape,
        )(xaug, sumb, st1, w0f, w0s4, b0i, w1i, b1i)

        bout, sumb = pl.pallas_call(
            ph2,
            grid=(nsteps,),
            in_specs=[x_spec, full2((nbm, K)), full2((2, 64)),
                      full2((2, 64)), full2((4, 4, 64)), full2((4, 64)),
                      full2((1, 64)), full2((4, 64, 64)), full2((1, 64)),
                      full2((4, 64)), full2((1, 1))],
            out_specs=[full2((nbm, K)), full2((nbm, K))],
            out_shape=[jax.ShapeDtypeStruct((nbm, K), f32),
                       jax.ShapeDtypeStruct((nbm, K), f32)],
            scratch_shapes=[pltpu.VMEM((nbm, K), f32)],
        )(xaug, sumb, st1, st2, w0f, w0s4, b0i, w1i, b1i, w2i, b2i)
        bs.append(bout)

    bb = jnp.stack(bs, axis=0)
    bbh = jnp.transpose(bb, (1, 0, 2)).reshape(B, M, NRF, K)
    ah = jnp.transpose(A, (0, 2, 1, 3, 4, 5)).reshape(B, M, C, K, P)

    head = _make_head(M, K, P)
    hnew5, x5, zb = pl.pallas_call(
        head,
        grid=(B,),
        in_specs=[pl.BlockSpec((1, M, C, K, P), lambda b: (b, 0, 0, 0, 0)),
                  pl.BlockSpec((1, M, NRF, K), lambda b: (b, 0, 0, 0))],
        out_specs=[pl.BlockSpec((1, M, C, NRF, P), lambda b: (b, 0, 0, 0, 0)),
                   pl.BlockSpec((1, M, C, NRF, P), lambda b: (b, 0, 0, 0, 0)),
                   pl.BlockSpec((B, 1), lambda b: (0, 0))],
        out_shape=[jax.ShapeDtypeStruct((B, M, C, NRF, P), f32),
                   jax.ShapeDtypeStruct((B, M, C, NRF, P), f32),
                   jax.ShapeDtypeStruct((B, 1), f32)],
    )(ah, bbh)

    hnew = jnp.transpose(hnew5.reshape(B, M, C, NRF, NR, NT),
                         (0, 2, 1, 3, 4, 5))
    x_pred = jnp.transpose(x5.reshape(B, M, C, NRF, NR, NT),
                           (0, 2, 1, 3, 4, 5))
    y_pred = jnp.full((B, M), 1.0 / M, dtype=f32)
    z_pred = zb.reshape(B)
    return hnew, x_pred, y_pred, z_pred


# speedup vs baseline: 3.9859x; 1.5511x over previous
import jax
import jax.numpy as jnp
from jax import lax
from jax.experimental import pallas as pl
from jax.experimental.pallas import tpu as pltpu

_BN_EPS = 1e-5


def _prep_kernel(at_ref, x_ref):
    re = at_ref[0]
    im = at_ref[1]
    stre = jnp.sqrt(re * re + im * im + 1e-12)
    nrm = jnp.sqrt(jnp.sum(stre * stre, axis=1, keepdims=True) + 1e-12)
    x_ref[0] = re
    x_ref[1] = im
    x_ref[2] = stre
    x_ref[3] = stre / nrm


def _mu_rs(st, npos):
    mu = st[0] / npos
    var = st[1] / npos - mu * mu
    return mu, lax.rsqrt(var + _BN_EPS)


def _y0_block(x, sumb_blk, w0f, w0s, beff, cb, nr, nt):
    d = w0f.shape[-1]
    xr = x.reshape(4, cb, nr, nt)
    m_nr = jnp.mean(xr, axis=2)
    m_nt = jnp.mean(xr, axis=3)
    m_all = jnp.mean(m_nt, axis=2)
    cd = (((0,), (0,)), ((), ()))
    y = lax.dot_general(x.reshape(4, cb * nr * nt), w0f[0], cd,
                        preferred_element_type=jnp.float32)
    y = y.reshape(cb, nr, nt, d)
    ynr = lax.dot_general(m_nr.reshape(4, cb * nt), w0f[1], cd,
                          preferred_element_type=jnp.float32
                          ).reshape(cb, nt, d)
    ynt = lax.dot_general(m_nt.reshape(4, cb * nr), w0f[2], cd,
                          preferred_element_type=jnp.float32
                          ).reshape(cb, nr, d)
    percell = lax.dot_general(m_all, w0f[3], cd,
                              preferred_element_type=jnp.float32)
    ycorr = ynt + percell[:, None, :] + beff[None, None, :]
    ycorr = ycorr + sumb_blk[:, None, None] * w0s[None, None, :]
    y = y + ynr[:, None]
    y = y + ycorr[:, :, None, :]
    return y


def _layer1_block(h, w14, b1, cb, nr, nt):
    d = w14.shape[-1]
    m_nr = jnp.mean(h, axis=1)
    m_nt = jnp.mean(h, axis=2)
    m_all = jnp.mean(m_nt, axis=1)
    y = jnp.dot(h.reshape(cb * nr * nt, d), w14[0],
                preferred_element_type=jnp.float32).reshape(cb, nr, nt, d)
    ynr = jnp.dot(m_nr.reshape(cb * nt, d), w14[1],
                  preferred_element_type=jnp.float32).reshape(cb, nt, d)
    ynt = jnp.dot(m_nt.reshape(cb * nr, d), w14[2],
                  preferred_element_type=jnp.float32).reshape(cb, nr, d)
    percell = jnp.dot(m_all, w14[3],
                      preferred_element_type=jnp.float32) + b1[None, :]
    y = y + ynr[:, None]
    y = y + (ynt + percell[:, None, :])[:, :, None, :]
    return y


def _accum_stats(st_ref, y, s):
    ps = jnp.stack([jnp.sum(y, axis=(0, 1, 2)),
                    jnp.sum(y * y, axis=(0, 1, 2))])

    @pl.when(s == 0)
    def _():
        st_ref[...] = ps

    @pl.when(s != 0)
    def _():
        st_ref[...] = st_ref[...] + ps


def _make_ph0(cb, nr, nt, k):
    def body(x_ref, sumb_ref, w0f_ref, w0s_ref, b0_ref, st_ref):
        s = pl.program_id(0)
        cbr = cb // k
        sumb_blk = sumb_ref[pl.ds(s * cbr, cbr), :].reshape(cb)
        w0s = jnp.sum(w0s_ref[...], axis=0)
        y = _y0_block(x_ref[...], sumb_blk, w0f_ref[...], w0s, b0_ref[0],
                      cb, nr, nt)
        _accum_stats(st_ref, y, s)
    return body


def _make_ph1(cb, nr, nt, k, npos):
    def body(x_ref, sumb_ref, st1_ref, w0f_ref, w0s_ref, b0_ref,
             w1_ref, b1_ref, st_ref, y1_ref):
        s = pl.program_id(0)
        cbr = cb // k
        sumb_blk = sumb_ref[pl.ds(s * cbr, cbr), :].reshape(cb)
        mu1, rs1 = _mu_rs(st1_ref[...], npos)
        w0f = w0f_ref[...] * rs1[None, None, :]
        w0s = jnp.sum(w0s_ref[...], axis=0) * rs1
        beff = (b0_ref[0] - mu1) * rs1
        y0 = _y0_block(x_ref[...], sumb_blk, w0f, w0s, beff, cb, nr, nt)
        h0 = jnp.maximum(y0, 0.0)
        y1 = _layer1_block(h0, w1_ref[...], b1_ref[0], cb, nr, nt)
        y1_ref[...] = y1
        _accum_stats(st_ref, y1, s)
    return body


def _make_ph2(cb, nr, nt, k, npos, nsteps):
    def body(y1_ref, sumb_ref, st2_ref, w2_ref, b2_ref,
             bout_ref, sumbn_ref, logit_ref):
        s = pl.program_id(0)
        cbr = cb // k
        mu2, rs2 = _mu_rs(st2_ref[...], npos)
        y1 = y1_ref[...]
        h1 = jnp.maximum(y1 * rs2[None, None, None, :]
                         - (mu2 * rs2)[None, None, None, :], 0.0)
        mbar = jnp.mean(h1, axis=(1, 2))
        w2s = jnp.sum(w2_ref[...], axis=0)
        ylog = jnp.dot(mbar, w2s[:, None],
                       preferred_element_type=jnp.float32)
        ylog = ylog + b2_ref[0, 0]
        logit_ref[pl.ds(s * cbr, cbr), :] = ylog.reshape(cbr, k)

        @pl.when(s == nsteps - 1)
        def _():
            logits = logit_ref[...]
            mx = jnp.max(logits, axis=1, keepdims=True)
            e = jnp.exp(logits - mx)
            p = e / jnp.sum(e, axis=1, keepdims=True)
            bout_ref[...] = p
            sumbn_ref[...] = sumb_ref[...] + p
    return body


def _make_head(m, k, p):
    def body(a_ref, bb_ref, hnew_ref, x_ref, z_ref):
        ablk = a_ref[0]
        bblk = bb_ref[0]
        hn = lax.dot_general(bblk, ablk, (((2,), (2,)), ((0,), (0,))),
                             preferred_element_type=jnp.float32)
        g = jnp.sum(hn * hn, axis=(2, 3))
        hnt = jnp.transpose(hn, (0, 2, 1, 3))
        hnew_ref[0] = hnt
        x_ref[0] = hnt / (jnp.sqrt(g)[:, None, :, None] + 1e-8)
        z_ref[pl.ds(pl.program_id(0), 1), :] = (
            jnp.sum(jnp.log2(1.0 + g)).reshape(1, 1))
    return body


def kernel(A, params):
    B, C, M, K, NR, NT = A.shape
    NRF = params["W0"].shape[0]
    P = NR * NT
    ncell = B * M * K
    nbm = B * M
    npos = float(ncell * P)
    cb = min(64, ncell)
    nsteps = ncell // cb
    f32 = jnp.float32

    at = jnp.transpose(A, (1, 0, 2, 3, 4, 5)).reshape(C, ncell, P)
    pcb = min(128, ncell)
    xaug = pl.pallas_call(
        _prep_kernel,
        grid=(ncell // pcb,),
        in_specs=[pl.BlockSpec((C, pcb, P), lambda s: (0, s, 0))],
        out_specs=pl.BlockSpec((4, pcb, P), lambda s: (0, s, 0)),
        out_shape=jax.ShapeDtypeStruct((4, ncell, P), f32),
    )(at)

    full2 = lambda shape: pl.BlockSpec(shape, lambda s: tuple(0 for _ in shape))
    x_spec = pl.BlockSpec((4, cb, P), lambda s: (0, s, 0))
    y1_spec = pl.BlockSpec((cb, NR, NT, 64), lambda s: (s, 0, 0, 0))
    st_shape = jax.ShapeDtypeStruct((2, 64), f32)

    ph0 = _make_ph0(cb, NR, NT, K)
    ph1 = _make_ph1(cb, NR, NT, K, npos)
    ph2 = _make_ph2(cb, NR, NT, K, npos, nsteps)

    W0, W1, W2 = params["W0"], params["W1"], params["W2"]
    b0a, b1a, b2a = params["b0"], params["b1"], params["b2"]

    sumb = jnp.zeros((nbm, K), f32)
    bs = []
    for i in range(NRF):
        w0f = W0[i, :, :4, :]
        w0s4 = W0[i, :, 4, :]
        b0i = b0a[i:i + 1]
        w1i = W1[i]
        b1i = b1a[i:i + 1]
        w2i = W2[i, :, :, 0]
        b2i = b2a[i:i + 1]

        st1 = pl.pallas_call(
            ph0,
            grid=(nsteps,),
            in_specs=[x_spec, full2((nbm, K)), full2((4, 4, 64)),
                      full2((4, 64)), full2((1, 64))],
            out_specs=full2((2, 64)),
            out_shape=st_shape,
        )(xaug, sumb, w0f, w0s4, b0i)

        st2, y1 = pl.pallas_call(
            ph1,
            grid=(nsteps,),
            in_specs=[x_spec, full2((nbm, K)), full2((2, 64)),
                      full2((4, 4, 64)), full2((4, 64)), full2((1, 64)),
                      full2((4, 64, 64)), full2((1, 64))],
            out_specs=[full2((2, 64)), y1_spec],
            out_shape=[st_shape,
                       jax.ShapeDtypeStruct((ncell, NR, NT, 64), f32)],
        )(xaug, sumb, st1, w0f, w0s4, b0i, w1i, b1i)

        bout, sumb = pl.pallas_call(
            ph2,
            grid=(nsteps,),
            in_specs=[y1_spec, full2((nbm, K)), full2((2, 64)),
                      full2((4, 64)), full2((1, 1))],
            out_specs=[full2((nbm, K)), full2((nbm, K))],
            out_shape=[jax.ShapeDtypeStruct((nbm, K), f32),
                       jax.ShapeDtypeStruct((nbm, K), f32)],
            scratch_shapes=[pltpu.VMEM((nbm, K), f32)],
        )(y1, sumb, st2, w2i, b2i)
        bs.append(bout)

    bb = jnp.stack(bs, axis=0)
    bbh = jnp.transpose(bb, (1, 0, 2)).reshape(B, M, NRF, K)
    ah = jnp.transpose(A, (0, 2, 1, 3, 4, 5)).reshape(B, M, C, K, P)

    head = _make_head(M, K, P)
    hnew5, x5, zb = pl.pallas_call(
        head,
        grid=(B,),
        in_specs=[pl.BlockSpec((1, M, C, K, P), lambda b: (b, 0, 0, 0, 0)),
                  pl.BlockSpec((1, M, NRF, K), lambda b: (b, 0, 0, 0))],
        out_specs=[pl.BlockSpec((1, M, C, NRF, P), lambda b: (b, 0, 0, 0, 0)),
                   pl.BlockSpec((1, M, C, NRF, P), lambda b: (b, 0, 0, 0, 0)),
                   pl.BlockSpec((B, 1), lambda b: (0, 0))],
        out_shape=[jax.ShapeDtypeStruct((B, M, C, NRF, P), f32),
                   jax.ShapeDtypeStruct((B, M, C, NRF, P), f32),
                   jax.ShapeDtypeStruct((B, 1), f32)],
    )(ah, bbh)

    hnew = jnp.transpose(hnew5.reshape(B, M, C, NRF, NR, NT),
                         (0, 2, 1, 3, 4, 5))
    x_pred = jnp.transpose(x5.reshape(B, M, C, NRF, NR, NT),
                           (0, 2, 1, 3, 4, 5))
    y_pred = jnp.full((B, M), 1.0 / M, dtype=f32)
    z_pred = zb.reshape(B)
    return hnew, x_pred, y_pred, z_pred


# speedup vs baseline: 6.0222x; 1.5109x over previous
import jax
import jax.numpy as jnp
from jax import lax
from jax.experimental import pallas as pl
from jax.experimental.pallas import tpu as pltpu

_BN_EPS = 1e-5


def _make_prep(pcb, nr, nt):
    def body(at_ref, x_ref, cm_ref, gram_ref, scm_ref):
        s = pl.program_id(0)
        re = at_ref[0]
        im = at_ref[1]
        stre = jnp.sqrt(re * re + im * im + 1e-12)
        nrm = jnp.sqrt(jnp.sum(stre * stre, axis=1, keepdims=True) + 1e-12)
        x_ref[0] = re
        x_ref[1] = im
        x_ref[2] = stre
        x_ref[3] = stre / nrm
        x4 = x_ref[...]
        xr = x4.reshape(4, pcb, nr, nt)
        m_nr = jnp.mean(xr, axis=2)
        m_nt = jnp.mean(xr, axis=3)
        m_all = jnp.mean(m_nt, axis=2)
        cm_ref[...] = m_all
        cd = (((1,), (1,)), ((), ()))
        gx = lax.dot_general(x4.reshape(4, pcb * nr * nt),
                             x4.reshape(4, pcb * nr * nt), cd,
                             preferred_element_type=jnp.float32)
        gnr = lax.dot_general(m_nr.reshape(4, pcb * nt),
                              m_nr.reshape(4, pcb * nt), cd,
                              preferred_element_type=jnp.float32)
        gnt = lax.dot_general(m_nt.reshape(4, pcb * nr),
                              m_nt.reshape(4, pcb * nr), cd,
                              preferred_element_type=jnp.float32)
        gall = lax.dot_general(m_all, m_all, cd,
                               preferred_element_type=jnp.float32)
        gpart = jnp.stack([gx, gnr, gnt, gall])
        spart = jnp.sum(m_all, axis=1, keepdims=True)

        @pl.when(s == 0)
        def _():
            gram_ref[...] = gpart
            scm_ref[...] = spart

        @pl.when(s != 0)
        def _():
            gram_ref[...] = gram_ref[...] + gpart
            scm_ref[...] = scm_ref[...] + spart
    return body


def _make_bn0(npos, ncell, p):
    def body(sumb_ref, cm_ref, gram_ref, scm_ref, w0f_ref, w0s_ref,
             b0_ref, st_ref):
        sb = sumb_ref[...]
        cm = cm_ref[...]
        g = gram_ref[...]
        scm = scm_ref[...]
        w = w0f_ref[...]
        w0s = jnp.sum(w0s_ref[...], axis=0)
        b0 = b0_ref[0]
        ws = w[0] + w[1] + w[2] + w[3]

        ssum = jnp.sum(sb)
        ss2 = jnp.sum(sb * sb)
        svm = jnp.sum(jnp.sum(cm * sb[None], axis=2), axis=1,
                      keepdims=True)

        cdt = (((0,), (0,)), ((), ()))

        def dterm(wa, wb, gm):
            gwb = lax.dot_general(gm, wb, (((1,), (0,)), ((), ())),
                                  preferred_element_type=jnp.float32)
            return jnp.sum(wa * gwb, axis=0)

        scm_ws = lax.dot_general(scm, ws, cdt,
                                 preferred_element_type=jnp.float32)[0]
        svm_ws = lax.dot_general(svm, ws, cdt,
                                 preferred_element_type=jnp.float32)[0]

        mean_vec = p * scm_ws + npos * b0 + (p * ssum) * w0s
        sq = (dterm(w[0], w[0], g[0])
              + 4.0 * dterm(w[1], w[1], g[1])
              + 64.0 * dterm(w[2], w[2], g[2])
              + 256.0 * dterm(w[3], w[3], g[3])
              + 8.0 * dterm(w[0], w[1], g[1])
              + 128.0 * dterm(w[0], w[2], g[2])
              + 512.0 * (dterm(w[0], w[3], g[3])
                         + dterm(w[1], w[2], g[3])
                         + dterm(w[1], w[3], g[3])
                         + dterm(w[2], w[3], g[3]))
              + p * (ncell * b0 * b0 + 2.0 * ssum * b0 * w0s
                     + ss2 * w0s * w0s)
              + 2.0 * p * (b0 * scm_ws + w0s * svm_ws))
        st_ref[...] = jnp.stack([mean_vec, sq])
    return body


def _mu_rs(st, npos):
    mu = st[0] / npos
    var = st[1] / npos - mu * mu
    return mu, lax.rsqrt(var + _BN_EPS)


def _y0_block(x, sumb_blk, w0f, w0s, beff, cb, nr, nt):
    d = w0f.shape[-1]
    xr = x.reshape(4, cb, nr, nt)
    m_nr = jnp.mean(xr, axis=2)
    m_nt = jnp.mean(xr, axis=3)
    m_all = jnp.mean(m_nt, axis=2)
    cd = (((0,), (0,)), ((), ()))
    y = lax.dot_general(x.reshape(4, cb * nr * nt), w0f[0], cd,
                        preferred_element_type=jnp.float32)
    y = y.reshape(cb, nr, nt, d)
    ynr = lax.dot_general(m_nr.reshape(4, cb * nt), w0f[1], cd,
                          preferred_element_type=jnp.float32
                          ).reshape(cb, nt, d)
    ynt = lax.dot_general(m_nt.reshape(4, cb * nr), w0f[2], cd,
                          preferred_element_type=jnp.float32
                          ).reshape(cb, nr, d)
    percell = lax.dot_general(m_all, w0f[3], cd,
                              preferred_element_type=jnp.float32)
    ycorr = ynt + percell[:, None, :] + beff[None, None, :]
    ycorr = ycorr + sumb_blk[:, None, None] * w0s[None, None, :]
    y = y + ynr[:, None]
    y = y + ycorr[:, :, None, :]
    return y


def _layer1_block(h, w14, b1, cb, nr, nt):
    d = w14.shape[-1]
    m_nr = jnp.mean(h, axis=1)
    m_nt = jnp.mean(h, axis=2)
    m_all = jnp.mean(m_nt, axis=1)
    y = jnp.dot(h.reshape(cb * nr * nt, d), w14[0],
                preferred_element_type=jnp.float32).reshape(cb, nr, nt, d)
    ynr = jnp.dot(m_nr.reshape(cb * nt, d), w14[1],
                  preferred_element_type=jnp.float32).reshape(cb, nt, d)
    ynt = jnp.dot(m_nt.reshape(cb * nr, d), w14[2],
                  preferred_element_type=jnp.float32).reshape(cb, nr, d)
    percell = jnp.dot(m_all, w14[3],
                      preferred_element_type=jnp.float32) + b1[None, :]
    y = y + ynr[:, None]
    y = y + (ynt + percell[:, None, :])[:, :, None, :]
    return y


def _make_ph1(cb, nr, nt, k, npos):
    def body(x_ref, sumb_ref, st1_ref, w0f_ref, w0s_ref, b0_ref,
             w1_ref, b1_ref, st_ref, y1_ref):
        s = pl.program_id(0)
        cbr = cb // k
        sumb_blk = sumb_ref[pl.ds(s * cbr, cbr), :].reshape(cb)
        mu1, rs1 = _mu_rs(st1_ref[...], npos)
        w0f = w0f_ref[...] * rs1[None, None, :]
        w0s = jnp.sum(w0s_ref[...], axis=0) * rs1
        beff = (b0_ref[0] - mu1) * rs1
        y0 = _y0_block(x_ref[...], sumb_blk, w0f, w0s, beff, cb, nr, nt)
        h0 = jnp.maximum(y0, 0.0)
        y1 = _layer1_block(h0, w1_ref[...], b1_ref[0], cb, nr, nt)
        y1_ref[...] = y1
        d = y1.shape[-1]
        y1f = y1.reshape(cb * nr * nt, d)
        gram = lax.dot_general(y1f, y1f, (((0,), (0,)), ((), ())),
                               preferred_element_type=jnp.float32)
        eye = (lax.broadcasted_iota(jnp.int32, (d, d), 0)
               == lax.broadcasted_iota(jnp.int32, (d, d), 1))
        sq_vec = jnp.sum(jnp.where(eye, gram, 0.0), axis=1)
        sum_vec = jnp.sum(y1, axis=(0, 1, 2))
        ps = jnp.stack([sum_vec, sq_vec])

        @pl.when(s == 0)
        def _():
            st_ref[...] = ps

        @pl.when(s != 0)
        def _():
            st_ref[...] = st_ref[...] + ps
    return body


def _make_ph2(cb, nr, nt, k, npos, nsteps):
    def body(y1_ref, sumb_ref, st2_ref, w2_ref, b2_ref,
             bout_ref, sumbn_ref, logit_ref):
        s = pl.program_id(0)
        cbr = cb // k
        mu2, rs2 = _mu_rs(st2_ref[...], npos)
        y1 = y1_ref[...]
        h1 = jnp.maximum(y1 * rs2[None, None, None, :]
                         - (mu2 * rs2)[None, None, None, :], 0.0)
        mbar = jnp.mean(h1, axis=(1, 2))
        w2s = jnp.sum(w2_ref[...], axis=0)
        ylog = jnp.dot(mbar, w2s[:, None],
                       preferred_element_type=jnp.float32)
        ylog = ylog + b2_ref[0, 0]
        logit_ref[pl.ds(s * cbr, cbr), :] = ylog.reshape(cbr, k)

        @pl.when(s == nsteps - 1)
        def _():
            logits = logit_ref[...]
            mx = jnp.max(logits, axis=1, keepdims=True)
            e = jnp.exp(logits - mx)
            p = e / jnp.sum(e, axis=1, keepdims=True)
            bout_ref[...] = p
            sumbn_ref[...] = sumb_ref[...] + p
    return body


def _make_head(m, k, p):
    def body(a_ref, bb_ref, hnew_ref, x_ref, z_ref):
        ablk = a_ref[0]
        bblk = bb_ref[0]
        hn = lax.dot_general(bblk, ablk, (((2,), (2,)), ((0,), (0,))),
                             preferred_element_type=jnp.float32)
        g = jnp.sum(hn * hn, axis=(2, 3))
        hnt = jnp.transpose(hn, (0, 2, 1, 3))
        hnew_ref[0] = hnt
        x_ref[0] = hnt / (jnp.sqrt(g)[:, None, :, None] + 1e-8)
        z_ref[pl.ds(pl.program_id(0), 1), :] = (
            jnp.sum(jnp.log2(1.0 + g)).reshape(1, 1))
    return body


def kernel(A, params):
    B, C, M, K, NR, NT = A.shape
    NRF = params["W0"].shape[0]
    P = NR * NT
    ncell = B * M * K
    nbm = B * M
    npos = float(ncell * P)
    cb = min(64, ncell)
    nsteps = ncell // cb
    f32 = jnp.float32

    at = jnp.transpose(A, (1, 0, 2, 3, 4, 5)).reshape(C, ncell, P)
    pcb = min(128, ncell)
    prep = _make_prep(pcb, NR, NT)
    xaug, cellmean, grams, scm = pl.pallas_call(
        prep,
        grid=(ncell // pcb,),
        in_specs=[pl.BlockSpec((C, pcb, P), lambda s: (0, s, 0))],
        out_specs=[pl.BlockSpec((4, pcb, P), lambda s: (0, s, 0)),
                   pl.BlockSpec((4, pcb), lambda s: (0, s)),
                   pl.BlockSpec((4, 4, 4), lambda s: (0, 0, 0)),
                   pl.BlockSpec((4, 1), lambda s: (0, 0))],
        out_shape=[jax.ShapeDtypeStruct((4, ncell, P), f32),
                   jax.ShapeDtypeStruct((4, ncell), f32),
                   jax.ShapeDtypeStruct((4, 4, 4), f32),
                   jax.ShapeDtypeStruct((4, 1), f32)],
    )(at)
    cm3 = cellmean.reshape(4, nbm, K)

    full2 = lambda shape: pl.BlockSpec(shape, lambda s: tuple(0 for _ in shape))
    x_spec = pl.BlockSpec((4, cb, P), lambda s: (0, s, 0))
    y1_spec = pl.BlockSpec((cb, NR, NT, 64), lambda s: (s, 0, 0, 0))
    st_shape = jax.ShapeDtypeStruct((2, 64), f32)

    bn0 = _make_bn0(npos, float(ncell), float(P))
    ph1 = _make_ph1(cb, NR, NT, K, npos)
    ph2 = _make_ph2(cb, NR, NT, K, npos, nsteps)

    W0, W1, W2 = params["W0"], params["W1"], params["W2"]
    b0a, b1a, b2a = params["b0"], params["b1"], params["b2"]

    sumb = jnp.zeros((nbm, K), f32)
    bs = []
    for i in range(NRF):
        w0f = W0[i, :, :4, :]
        w0s4 = W0[i, :, 4, :]
        b0i = b0a[i:i + 1]
        w1i = W1[i]
        b1i = b1a[i:i + 1]
        w2i = W2[i, :, :, 0]
        b2i = b2a[i:i + 1]

        st1 = pl.pallas_call(
            bn0,
            grid=(1,),
            in_specs=[full2((nbm, K)), full2((4, nbm, K)), full2((4, 4, 4)),
                      full2((4, 1)), full2((4, 4, 64)), full2((4, 64)),
                      full2((1, 64))],
            out_specs=full2((2, 64)),
            out_shape=st_shape,
        )(sumb, cm3, grams, scm, w0f, w0s4, b0i)

        st2, y1 = pl.pallas_call(
            ph1,
            grid=(nsteps,),
            in_specs=[x_spec, full2((nbm, K)), full2((2, 64)),
                      full2((4, 4, 64)), full2((4, 64)), full2((1, 64)),
                      full2((4, 64, 64)), full2((1, 64))],
            out_specs=[full2((2, 64)), y1_spec],
            out_shape=[st_shape,
                       jax.ShapeDtypeStruct((ncell, NR, NT, 64), f32)],
        )(xaug, sumb, st1, w0f, w0s4, b0i, w1i, b1i)

        bout, sumb = pl.pallas_call(
            ph2,
            grid=(nsteps,),
            in_specs=[y1_spec, full2((nbm, K)), full2((2, 64)),
                      full2((4, 64)), full2((1, 1))],
            out_specs=[full2((nbm, K)), full2((nbm, K))],
            out_shape=[jax.ShapeDtypeStruct((nbm, K), f32),
                       jax.ShapeDtypeStruct((nbm, K), f32)],
            scratch_shapes=[pltpu.VMEM((nbm, K), f32)],
        )(y1, sumb, st2, w2i, b2i)
        bs.append(bout)

    bb = jnp.stack(bs, axis=0)
    bbh = jnp.transpose(bb, (1, 0, 2)).reshape(B, M, NRF, K)
    ah = jnp.transpose(A, (0, 2, 1, 3, 4, 5)).reshape(B, M, C, K, P)

    head = _make_head(M, K, P)
    hnew5, x5, zb = pl.pallas_call(
        head,
        grid=(B,),
        in_specs=[pl.BlockSpec((1, M, C, K, P), lambda b: (b, 0, 0, 0, 0)),
                  pl.BlockSpec((1, M, NRF, K), lambda b: (b, 0, 0, 0))],
        out_specs=[pl.BlockSpec((1, M, C, NRF, P), lambda b: (b, 0, 0, 0, 0)),
                   pl.BlockSpec((1, M, C, NRF, P), lambda b: (b, 0, 0, 0, 0)),
                   pl.BlockSpec((B, 1), lambda b: (0, 0))],
        out_shape=[jax.ShapeDtypeStruct((B, M, C, NRF, P), f32),
                   jax.ShapeDtypeStruct((B, M, C, NRF, P), f32),
                   jax.ShapeDtypeStruct((B, 1), f32)],
    )(ah, bbh)

    hnew = jnp.transpose(hnew5.reshape(B, M, C, NRF, NR, NT),
                         (0, 2, 1, 3, 4, 5))
    x_pred = jnp.transpose(x5.reshape(B, M, C, NRF, NR, NT),
                           (0, 2, 1, 3, 4, 5))
    y_pred = jnp.full((B, M), 1.0 / M, dtype=f32)
    z_pred = zb.reshape(B)
    return hnew, x_pred, y_pred, z_pred


# speedup vs baseline: 8.1854x; 1.3592x over previous
import jax
import jax.numpy as jnp
from jax import lax
from jax.experimental import pallas as pl
from jax.experimental.pallas import tpu as pltpu

_BN_EPS = 1e-5


def _make_prep(pcb, nr, nt):
    def body(at_ref, x_ref, cm_ref, gram_ref, scm_ref):
        s = pl.program_id(0)
        re = at_ref[0]
        im = at_ref[1]
        stre = jnp.sqrt(re * re + im * im + 1e-12)
        nrm = jnp.sqrt(jnp.sum(stre * stre, axis=1, keepdims=True) + 1e-12)
        x_ref[0] = re
        x_ref[1] = im
        x_ref[2] = stre
        x_ref[3] = stre / nrm
        x4 = x_ref[0:4]
        xr = x4.reshape(4, pcb, nr, nt)
        m_nr = jnp.mean(xr, axis=2)
        m_nt = jnp.mean(xr, axis=3)
        m_all = jnp.mean(m_nt, axis=2)
        cm_ref[...] = jnp.transpose(m_all)
        for c in range(4):
            x_ref[4 + c] = jnp.broadcast_to(
                m_nr[c][:, None, :], (pcb, nr, nt)).reshape(pcb, nr * nt)
            x_ref[8 + c] = jnp.broadcast_to(
                m_nt[c][:, :, None], (pcb, nr, nt)).reshape(pcb, nr * nt)
        cd = (((1,), (1,)), ((), ()))
        gx = lax.dot_general(x4.reshape(4, pcb * nr * nt),
                             x4.reshape(4, pcb * nr * nt), cd,
                             preferred_element_type=jnp.float32)
        gnr = lax.dot_general(m_nr.reshape(4, pcb * nt),
                              m_nr.reshape(4, pcb * nt), cd,
                              preferred_element_type=jnp.float32)
        gnt = lax.dot_general(m_nt.reshape(4, pcb * nr),
                              m_nt.reshape(4, pcb * nr), cd,
                              preferred_element_type=jnp.float32)
        gall = lax.dot_general(m_all, m_all, cd,
                               preferred_element_type=jnp.float32)
        gpart = jnp.stack([gx, gnr, gnt, gall])
        spart = jnp.sum(m_all, axis=1, keepdims=True)

        @pl.when(s == 0)
        def _():
            gram_ref[...] = gpart
            scm_ref[...] = spart

        @pl.when(s != 0)
        def _():
            gram_ref[...] = gram_ref[...] + gpart
            scm_ref[...] = scm_ref[...] + spart
    return body


def _make_bn0(npos, ncell, p):
    def body(sumb_ref, cm_ref, gram_ref, scm_ref, w0f_ref, w0s_ref,
             b0_ref, st_ref):
        sb = sumb_ref[...]
        cm = cm_ref[...]
        g = gram_ref[...]
        scm = scm_ref[...]
        w = w0f_ref[...]
        w0s = jnp.sum(w0s_ref[...], axis=0)
        b0 = b0_ref[0]
        ws = w[0] + w[1] + w[2] + w[3]

        ssum = jnp.sum(sb)
        ss2 = jnp.sum(sb * sb)
        svm = jnp.sum(jnp.sum(cm * sb[None], axis=2), axis=1,
                      keepdims=True)

        cdt = (((0,), (0,)), ((), ()))

        def dterm(wa, wb, gm):
            gwb = lax.dot_general(gm, wb, (((1,), (0,)), ((), ())),
                                  preferred_element_type=jnp.float32)
            return jnp.sum(wa * gwb, axis=0)

        scm_ws = lax.dot_general(scm, ws, cdt,
                                 preferred_element_type=jnp.float32)[0]
        svm_ws = lax.dot_general(svm, ws, cdt,
                                 preferred_element_type=jnp.float32)[0]

        mean_vec = p * scm_ws + npos * b0 + (p * ssum) * w0s
        sq = (dterm(w[0], w[0], g[0])
              + 4.0 * dterm(w[1], w[1], g[1])
              + 64.0 * dterm(w[2], w[2], g[2])
              + 256.0 * dterm(w[3], w[3], g[3])
              + 8.0 * dterm(w[0], w[1], g[1])
              + 128.0 * dterm(w[0], w[2], g[2])
              + 512.0 * (dterm(w[0], w[3], g[3])
                         + dterm(w[1], w[2], g[3])
                         + dterm(w[1], w[3], g[3])
                         + dterm(w[2], w[3], g[3]))
              + p * (ncell * b0 * b0 + 2.0 * ssum * b0 * w0s
                     + ss2 * w0s * w0s)
              + 2.0 * p * (b0 * scm_ws + w0s * svm_ws))
        st_ref[...] = jnp.stack([mean_vec, sq])
    return body


def _mu_rs(st, npos):
    mu = st[0] / npos
    var = st[1] / npos - mu * mu
    return mu, lax.rsqrt(var + _BN_EPS)


def _layer1_block(h, w14, b1, cb, nr, nt):
    d = w14.shape[-1]
    m_nr = jnp.mean(h, axis=1)
    m_nt = jnp.mean(h, axis=2)
    m_all = jnp.mean(m_nt, axis=1)
    y = jnp.dot(h.reshape(cb * nr * nt, d), w14[0],
                preferred_element_type=jnp.float32).reshape(cb, nr, nt, d)
    ynr = jnp.dot(m_nr.reshape(cb * nt, d), w14[1],
                  preferred_element_type=jnp.float32).reshape(cb, nt, d)
    ynt = jnp.dot(m_nt.reshape(cb * nr, d), w14[2],
                  preferred_element_type=jnp.float32).reshape(cb, nr, d)
    percell = jnp.dot(m_all, w14[3],
                      preferred_element_type=jnp.float32) + b1[None, :]
    y = y + ynr[:, None]
    y = y + (ynt + percell[:, None, :])[:, :, None, :]
    return y


def _make_ph1(cb, nr, nt, k, npos):
    def body(x_ref, cm_ref, sumb_ref, st1_ref, w0f_ref, w0s_ref, b0_ref,
             w1_ref, b1_ref, st_ref, y1_ref):
        s = pl.program_id(0)
        cbr = cb // k
        sumb_blk = sumb_ref[pl.ds(s * cbr, cbr), :].reshape(cb)
        mu1, rs1 = _mu_rs(st1_ref[...], npos)
        w = w0f_ref[...]
        wcat = (jnp.concatenate([w[0], w[1], w[2]], axis=0)
                * rs1[None, :])
        w0s = jnp.sum(w0s_ref[...], axis=0) * rs1
        beff = (b0_ref[0] - mu1) * rs1
        cd = (((0,), (0,)), ((), ()))
        x12 = x_ref[...]
        y3 = lax.dot_general(x12.reshape(12, cb * nr * nt), wcat, cd,
                             preferred_element_type=jnp.float32
                             ).reshape(cb, nr * nt, 64)
        percell = lax.dot_general(cm_ref[...], w[3] * rs1[None, :],
                                  (((1,), (0,)), ((), ())),
                                  preferred_element_type=jnp.float32)
        ycorr = (percell[:, None, :] + beff[None, None, :]
                 + sumb_blk[:, None, None] * w0s[None, None, :])
        h0 = jnp.maximum(y3 + ycorr, 0.0).reshape(cb, nr, nt, 64)
        y1 = _layer1_block(h0, w1_ref[...], b1_ref[0], cb, nr, nt)
        y1_ref[...] = y1
        d = y1.shape[-1]
        y1f = y1.reshape(cb * nr * nt, d)
        gram = lax.dot_general(y1f, y1f, (((0,), (0,)), ((), ())),
                               preferred_element_type=jnp.float32)
        eye = (lax.broadcasted_iota(jnp.int32, (d, d), 0)
               == lax.broadcasted_iota(jnp.int32, (d, d), 1))
        sq_vec = jnp.sum(jnp.where(eye, gram, 0.0), axis=1)
        sum_vec = jnp.sum(y1, axis=(0, 1, 2))
        ps = jnp.stack([sum_vec, sq_vec])

        @pl.when(s == 0)
        def _():
            st_ref[...] = ps

        @pl.when(s != 0)
        def _():
            st_ref[...] = st_ref[...] + ps
    return body


def _make_ph2(cb, nr, nt, k, npos, nsteps):
    def body(y1_ref, sumb_ref, st2_ref, w2_ref, b2_ref,
             bout_ref, sumbn_ref, logit_ref):
        s = pl.program_id(0)
        cbr = cb // k
        mu2, rs2 = _mu_rs(st2_ref[...], npos)
        y1 = y1_ref[...]
        h1 = jnp.maximum(y1 * rs2[None, None, None, :]
                         - (mu2 * rs2)[None, None, None, :], 0.0)
        mbar = jnp.mean(h1, axis=(1, 2))
        w2s = jnp.sum(w2_ref[...], axis=0)
        ylog = jnp.dot(mbar, w2s[:, None],
                       preferred_element_type=jnp.float32)
        ylog = ylog + b2_ref[0, 0]
        logit_ref[pl.ds(s * cbr, cbr), :] = ylog.reshape(cbr, k)

        @pl.when(s == nsteps - 1)
        def _():
            logits = logit_ref[...]
            mx = jnp.max(logits, axis=1, keepdims=True)
            e = jnp.exp(logits - mx)
            p = e / jnp.sum(e, axis=1, keepdims=True)
            bout_ref[...] = p
            sumbn_ref[...] = sumb_ref[...] + p
    return body


def _make_head(m, k, p):
    def body(a_ref, bb_ref, hnew_ref, x_ref, z_ref):
        ablk = a_ref[0]
        bblk = bb_ref[0]
        hn = lax.dot_general(bblk, ablk, (((2,), (2,)), ((0,), (0,))),
                             preferred_element_type=jnp.float32)
        g = jnp.sum(hn * hn, axis=(2, 3))
        hnt = jnp.transpose(hn, (0, 2, 1, 3))
        hnew_ref[0] = hnt
        x_ref[0] = hnt / (jnp.sqrt(g)[:, None, :, None] + 1e-8)
        z_ref[pl.ds(pl.program_id(0), 1), :] = (
            jnp.sum(jnp.log2(1.0 + g)).reshape(1, 1))
    return body


def kernel(A, params):
    B, C, M, K, NR, NT = A.shape
    NRF = params["W0"].shape[0]
    P = NR * NT
    ncell = B * M * K
    nbm = B * M
    npos = float(ncell * P)
    cb = min(64, ncell)
    nsteps = ncell // cb
    f32 = jnp.float32

    at = jnp.transpose(A, (1, 0, 2, 3, 4, 5)).reshape(C, ncell, P)
    pcb = min(128, ncell)
    prep = _make_prep(pcb, NR, NT)
    xaug, cellmean, grams, scm = pl.pallas_call(
        prep,
        grid=(ncell // pcb,),
        in_specs=[pl.BlockSpec((C, pcb, P), lambda s: (0, s, 0))],
        out_specs=[pl.BlockSpec((12, pcb, P), lambda s: (0, s, 0)),
                   pl.BlockSpec((pcb, 4), lambda s: (s, 0)),
                   pl.BlockSpec((4, 4, 4), lambda s: (0, 0, 0)),
                   pl.BlockSpec((4, 1), lambda s: (0, 0))],
        out_shape=[jax.ShapeDtypeStruct((12, ncell, P), f32),
                   jax.ShapeDtypeStruct((ncell, 4), f32),
                   jax.ShapeDtypeStruct((4, 4, 4), f32),
                   jax.ShapeDtypeStruct((4, 1), f32)],
    )(at)
    cm3 = jnp.transpose(cellmean).reshape(4, nbm, K)

    full2 = lambda shape: pl.BlockSpec(shape, lambda s: tuple(0 for _ in shape))
    x_spec = pl.BlockSpec((12, cb, P), lambda s: (0, s, 0))
    cm_spec = pl.BlockSpec((cb, 4), lambda s: (s, 0))
    y1_spec = pl.BlockSpec((cb, NR, NT, 64), lambda s: (s, 0, 0, 0))
    st_shape = jax.ShapeDtypeStruct((2, 64), f32)

    bn0 = _make_bn0(npos, float(ncell), float(P))
    ph1 = _make_ph1(cb, NR, NT, K, npos)
    ph2 = _make_ph2(cb, NR, NT, K, npos, nsteps)

    W0, W1, W2 = params["W0"], params["W1"], params["W2"]
    b0a, b1a, b2a = params["b0"], params["b1"], params["b2"]

    sumb = jnp.zeros((nbm, K), f32)
    bs = []
    for i in range(NRF):
        w0f = W0[i, :, :4, :]
        w0s4 = W0[i, :, 4, :]
        b0i = b0a[i:i + 1]
        w1i = W1[i]
        b1i = b1a[i:i + 1]
        w2i = W2[i, :, :, 0]
        b2i = b2a[i:i + 1]

        st1 = pl.pallas_call(
            bn0,
            grid=(1,),
            in_specs=[full2((nbm, K)), full2((4, nbm, K)), full2((4, 4, 4)),
                      full2((4, 1)), full2((4, 4, 64)), full2((4, 64)),
                      full2((1, 64))],
            out_specs=full2((2, 64)),
            out_shape=st_shape,
        )(sumb, cm3, grams, scm, w0f, w0s4, b0i)

        st2, y1 = pl.pallas_call(
            ph1,
            grid=(nsteps,),
            in_specs=[x_spec, cm_spec, full2((nbm, K)), full2((2, 64)),
                      full2((4, 4, 64)), full2((4, 64)), full2((1, 64)),
                      full2((4, 64, 64)), full2((1, 64))],
            out_specs=[full2((2, 64)), y1_spec],
            out_shape=[st_shape,
                       jax.ShapeDtypeStruct((ncell, NR, NT, 64), f32)],
        )(xaug, cellmean, sumb, st1, w0f, w0s4, b0i, w1i, b1i)

        bout, sumb = pl.pallas_call(
            ph2,
            grid=(nsteps,),
            in_specs=[y1_spec, full2((nbm, K)), full2((2, 64)),
                      full2((4, 64)), full2((1, 1))],
            out_specs=[full2((nbm, K)), full2((nbm, K))],
            out_shape=[jax.ShapeDtypeStruct((nbm, K), f32),
                       jax.ShapeDtypeStruct((nbm, K), f32)],
            scratch_shapes=[pltpu.VMEM((nbm, K), f32)],
        )(y1, sumb, st2, w2i, b2i)
        bs.append(bout)

    bb = jnp.stack(bs, axis=0)
    bbh = jnp.transpose(bb, (1, 0, 2)).reshape(B, M, NRF, K)
    ah = jnp.transpose(A, (0, 2, 1, 3, 4, 5)).reshape(B, M, C, K, P)

    head = _make_head(M, K, P)
    hnew5, x5, zb = pl.pallas_call(
        head,
        grid=(B,),
        in_specs=[pl.BlockSpec((1, M, C, K, P), lambda b: (b, 0, 0, 0, 0)),
                  pl.BlockSpec((1, M, NRF, K), lambda b: (b, 0, 0, 0))],
        out_specs=[pl.BlockSpec((1, M, C, NRF, P), lambda b: (b, 0, 0, 0, 0)),
                   pl.BlockSpec((1, M, C, NRF, P), lambda b: (b, 0, 0, 0, 0)),
                   pl.BlockSpec((B, 1), lambda b: (0, 0))],
        out_shape=[jax.ShapeDtypeStruct((B, M, C, NRF, P), f32),
                   jax.ShapeDtypeStruct((B, M, C, NRF, P), f32),
                   jax.ShapeDtypeStruct((B, 1), f32)],
    )(ah, bbh)

    hnew = jnp.transpose(hnew5.reshape(B, M, C, NRF, NR, NT),
                         (0, 2, 1, 3, 4, 5))
    x_pred = jnp.transpose(x5.reshape(B, M, C, NRF, NR, NT),
                           (0, 2, 1, 3, 4, 5))
    y_pred = jnp.full((B, M), 1.0 / M, dtype=f32)
    z_pred = zb.reshape(B)
    return hnew, x_pred, y_pred, z_pred


# speedup vs baseline: 9.5800x; 1.1704x over previous
import jax
import jax.numpy as jnp
from jax import lax
from jax.experimental import pallas as pl
from jax.experimental.pallas import tpu as pltpu

_BN_EPS = 1e-5


def _make_prep(pcb, nr, nt):
    def body(at_ref, x_ref, cm_ref, gram_ref, scm_ref):
        s = pl.program_id(0)
        re = at_ref[0]
        im = at_ref[1]
        stre = jnp.sqrt(re * re + im * im + 1e-12)
        nrm = jnp.sqrt(jnp.sum(stre * stre, axis=1, keepdims=True) + 1e-12)
        x_ref[0] = re
        x_ref[1] = im
        x_ref[2] = stre
        x_ref[3] = stre / nrm
        x4 = x_ref[0:4]
        xr = x4.reshape(4, pcb, nr, nt)
        m_nr = jnp.mean(xr, axis=2)
        m_nt = jnp.mean(xr, axis=3)
        m_all = jnp.mean(m_nt, axis=2)
        cm_ref[...] = jnp.transpose(m_all)
        for c in range(4):
            x_ref[4 + c] = jnp.broadcast_to(
                m_nr[c][:, None, :], (pcb, nr, nt)).reshape(pcb, nr * nt)
            x_ref[8 + c] = jnp.broadcast_to(
                m_nt[c][:, :, None], (pcb, nr, nt)).reshape(pcb, nr * nt)
        cd = (((1,), (1,)), ((), ()))
        gx = lax.dot_general(x4.reshape(4, pcb * nr * nt),
                             x4.reshape(4, pcb * nr * nt), cd,
                             preferred_element_type=jnp.float32)
        gnr = lax.dot_general(m_nr.reshape(4, pcb * nt),
                              m_nr.reshape(4, pcb * nt), cd,
                              preferred_element_type=jnp.float32)
        gnt = lax.dot_general(m_nt.reshape(4, pcb * nr),
                              m_nt.reshape(4, pcb * nr), cd,
                              preferred_element_type=jnp.float32)
        gall = lax.dot_general(m_all, m_all, cd,
                               preferred_element_type=jnp.float32)
        gpart = jnp.stack([gx, gnr, gnt, gall])
        spart = jnp.sum(m_all, axis=1, keepdims=True)

        @pl.when(s == 0)
        def _():
            gram_ref[...] = gpart
            scm_ref[...] = spart

        @pl.when(s != 0)
        def _():
            gram_ref[...] = gram_ref[...] + gpart
            scm_ref[...] = scm_ref[...] + spart
    return body


def _make_bn0(npos, ncell, p):
    def body(sumb_ref, cm_ref, gram_ref, scm_ref, w0f_ref, w0s_ref,
             b0_ref, st_ref):
        sb = sumb_ref[...]
        cm = cm_ref[...]
        g = gram_ref[...]
        scm = scm_ref[...]
        w = w0f_ref[...]
        w0s = jnp.sum(w0s_ref[...], axis=0)
        b0 = b0_ref[0]
        ws = w[0] + w[1] + w[2] + w[3]

        ssum = jnp.sum(sb)
        ss2 = jnp.sum(sb * sb)
        svm = jnp.sum(jnp.sum(cm * sb[None], axis=2), axis=1,
                      keepdims=True)

        cdt = (((0,), (0,)), ((), ()))

        def dterm(wa, wb, gm):
            gwb = lax.dot_general(gm, wb, (((1,), (0,)), ((), ())),
                                  preferred_element_type=jnp.float32)
            return jnp.sum(wa * gwb, axis=0)

        scm_ws = lax.dot_general(scm, ws, cdt,
                                 preferred_element_type=jnp.float32)[0]
        svm_ws = lax.dot_general(svm, ws, cdt,
                                 preferred_element_type=jnp.float32)[0]

        mean_vec = p * scm_ws + npos * b0 + (p * ssum) * w0s
        sq = (dterm(w[0], w[0], g[0])
              + 4.0 * dterm(w[1], w[1], g[1])
              + 64.0 * dterm(w[2], w[2], g[2])
              + 256.0 * dterm(w[3], w[3], g[3])
              + 8.0 * dterm(w[0], w[1], g[1])
              + 128.0 * dterm(w[0], w[2], g[2])
              + 512.0 * (dterm(w[0], w[3], g[3])
                         + dterm(w[1], w[2], g[3])
                         + dterm(w[1], w[3], g[3])
                         + dterm(w[2], w[3], g[3]))
              + p * (ncell * b0 * b0 + 2.0 * ssum * b0 * w0s
                     + ss2 * w0s * w0s)
              + 2.0 * p * (b0 * scm_ws + w0s * svm_ws))
        st_ref[...] = jnp.stack([mean_vec, sq])
    return body


def _mu_rs(st, npos):
    mu = st[0] / npos
    var = st[1] / npos - mu * mu
    return mu, lax.rsqrt(var + _BN_EPS)


def _layer1_block(h, hb, w14, b1, cb, nr, nt):
    d = w14.shape[-1]
    m_nr = jnp.mean(h, axis=1)
    m_nt = jnp.mean(h, axis=2)
    m_all = jnp.mean(m_nt, axis=1)
    bf = jnp.bfloat16
    y = jnp.dot(hb.reshape(cb * nr * nt, d), w14[0],
                preferred_element_type=jnp.float32).reshape(cb, nr, nt, d)
    ynr = jnp.dot(m_nr.reshape(cb * nt, d).astype(bf), w14[1],
                  preferred_element_type=jnp.float32).reshape(cb, nt, d)
    ynt = jnp.dot(m_nt.reshape(cb * nr, d).astype(bf), w14[2],
                  preferred_element_type=jnp.float32).reshape(cb, nr, d)
    percell = jnp.dot(m_all.astype(bf), w14[3],
                      preferred_element_type=jnp.float32) + b1[None, :]
    y = y + ynr[:, None]
    y = y + (ynt + percell[:, None, :])[:, :, None, :]
    return y


def _make_ph1(cb, nr, nt, k, npos):
    def body(x_ref, cm_ref, sumb_ref, st1_ref, w0f_ref, w0s_ref, b0_ref,
             w1_ref, b1_ref, st_ref, y1_ref):
        s = pl.program_id(0)
        cbr = cb // k
        sumb_blk = sumb_ref[pl.ds(s * cbr, cbr), :].reshape(cb)
        mu1, rs1 = _mu_rs(st1_ref[...], npos)
        bf = jnp.bfloat16
        w = w0f_ref[...]
        wcat = (jnp.concatenate([w[0], w[1], w[2]], axis=0)
                * rs1[None, :]).astype(bf)
        w0s = jnp.sum(w0s_ref[...], axis=0) * rs1
        beff = (b0_ref[0] - mu1) * rs1
        cd = (((0,), (0,)), ((), ()))
        x12 = x_ref[...].astype(bf)
        y3 = lax.dot_general(x12.reshape(12, cb * nr * nt), wcat, cd,
                             preferred_element_type=jnp.float32
                             ).reshape(cb, nr * nt, 64)
        percell = lax.dot_general(cm_ref[...], w[3] * rs1[None, :],
                                  (((1,), (0,)), ((), ())),
                                  preferred_element_type=jnp.float32)
        ycorr = (percell[:, None, :] + beff[None, None, :]
                 + sumb_blk[:, None, None] * w0s[None, None, :])
        h0 = jnp.maximum(y3 + ycorr, 0.0).reshape(cb, nr, nt, 64)
        y1 = _layer1_block(h0, h0.astype(bf), w1_ref[...].astype(bf),
                           b1_ref[0], cb, nr, nt)
        d = y1.shape[-1]
        y1b = y1.astype(bf)
        y1_ref[...] = y1b
        y1f = y1b.reshape(cb * nr * nt, d)
        gram = lax.dot_general(y1f, y1f, (((0,), (0,)), ((), ())),
                               preferred_element_type=jnp.float32)
        eye = (lax.broadcasted_iota(jnp.int32, (d, d), 0)
               == lax.broadcasted_iota(jnp.int32, (d, d), 1))
        sq_vec = jnp.sum(jnp.where(eye, gram, 0.0), axis=1)
        ones = jnp.ones((1, cb * nr * nt), dtype=bf)
        sum_vec = lax.dot_general(ones, y1f, (((1,), (0,)), ((), ())),
                                  preferred_element_type=jnp.float32)[0]
        ps = jnp.stack([sum_vec, sq_vec])

        @pl.when(s == 0)
        def _():
            st_ref[...] = ps

        @pl.when(s != 0)
        def _():
            st_ref[...] = st_ref[...] + ps
    return body


def _make_ph2(cb, nr, nt, k, npos, nsteps):
    def body(y1_ref, sumb_ref, st2_ref, w2_ref, b2_ref,
             bout_ref, sumbn_ref, logit_ref):
        s = pl.program_id(0)
        cbr = cb // k
        mu2, rs2 = _mu_rs(st2_ref[...], npos)
        y1 = y1_ref[...].astype(jnp.float32)
        h1 = jnp.maximum(y1 * rs2[None, None, None, :]
                         - (mu2 * rs2)[None, None, None, :], 0.0)
        mbar = jnp.mean(h1, axis=(1, 2))
        w2s = jnp.sum(w2_ref[...], axis=0)
        ylog = jnp.dot(mbar, w2s[:, None],
                       preferred_element_type=jnp.float32)
        ylog = ylog + b2_ref[0, 0]
        logit_ref[pl.ds(s * cbr, cbr), :] = ylog.reshape(cbr, k)

        @pl.when(s == nsteps - 1)
        def _():
            logits = logit_ref[...]
            mx = jnp.max(logits, axis=1, keepdims=True)
            e = jnp.exp(logits - mx)
            p = e / jnp.sum(e, axis=1, keepdims=True)
            bout_ref[...] = p
            sumbn_ref[...] = sumb_ref[...] + p
    return body


def _make_head(m, k, p):
    def body(a_ref, bb_ref, hnew_ref, x_ref, z_ref):
        ablk = a_ref[0]
        bblk = bb_ref[0]
        hn = lax.dot_general(bblk, ablk, (((2,), (2,)), ((0,), (0,))),
                             preferred_element_type=jnp.float32)
        g = jnp.sum(hn * hn, axis=(2, 3))
        hnt = jnp.transpose(hn, (0, 2, 1, 3))
        hnew_ref[0] = hnt
        x_ref[0] = hnt / (jnp.sqrt(g)[:, None, :, None] + 1e-8)
        z_ref[pl.ds(pl.program_id(0), 1), :] = (
            jnp.sum(jnp.log2(1.0 + g)).reshape(1, 1))
    return body


def kernel(A, params):
    B, C, M, K, NR, NT = A.shape
    NRF = params["W0"].shape[0]
    P = NR * NT
    ncell = B * M * K
    nbm = B * M
    npos = float(ncell * P)
    cb = min(64, ncell)
    nsteps = ncell // cb
    f32 = jnp.float32

    at = jnp.transpose(A, (1, 0, 2, 3, 4, 5)).reshape(C, ncell, P)
    pcb = min(128, ncell)
    prep = _make_prep(pcb, NR, NT)
    xaug, cellmean, grams, scm = pl.pallas_call(
        prep,
        grid=(ncell // pcb,),
        in_specs=[pl.BlockSpec((C, pcb, P), lambda s: (0, s, 0))],
        out_specs=[pl.BlockSpec((12, pcb, P), lambda s: (0, s, 0)),
                   pl.BlockSpec((pcb, 4), lambda s: (s, 0)),
                   pl.BlockSpec((4, 4, 4), lambda s: (0, 0, 0)),
                   pl.BlockSpec((4, 1), lambda s: (0, 0))],
        out_shape=[jax.ShapeDtypeStruct((12, ncell, P), f32),
                   jax.ShapeDtypeStruct((ncell, 4), f32),
                   jax.ShapeDtypeStruct((4, 4, 4), f32),
                   jax.ShapeDtypeStruct((4, 1), f32)],
    )(at)
    cm3 = jnp.transpose(cellmean).reshape(4, nbm, K)

    full2 = lambda shape: pl.BlockSpec(shape, lambda s: tuple(0 for _ in shape))
    x_spec = pl.BlockSpec((12, cb, P), lambda s: (0, s, 0))
    cm_spec = pl.BlockSpec((cb, 4), lambda s: (s, 0))
    y1_spec = pl.BlockSpec((cb, NR, NT, 64), lambda s: (s, 0, 0, 0))
    st_shape = jax.ShapeDtypeStruct((2, 64), f32)

    bn0 = _make_bn0(npos, float(ncell), float(P))
    ph1 = _make_ph1(cb, NR, NT, K, npos)
    ph2 = _make_ph2(cb, NR, NT, K, npos, nsteps)

    W0, W1, W2 = params["W0"], params["W1"], params["W2"]
    b0a, b1a, b2a = params["b0"], params["b1"], params["b2"]

    sumb = jnp.zeros((nbm, K), f32)
    bs = []
    for i in range(NRF):
        w0f = W0[i, :, :4, :]
        w0s4 = W0[i, :, 4, :]
        b0i = b0a[i:i + 1]
        w1i = W1[i]
        b1i = b1a[i:i + 1]
        w2i = W2[i, :, :, 0]
        b2i = b2a[i:i + 1]

        st1 = pl.pallas_call(
            bn0,
            grid=(1,),
            in_specs=[full2((nbm, K)), full2((4, nbm, K)), full2((4, 4, 4)),
                      full2((4, 1)), full2((4, 4, 64)), full2((4, 64)),
                      full2((1, 64))],
            out_specs=full2((2, 64)),
            out_shape=st_shape,
        )(sumb, cm3, grams, scm, w0f, w0s4, b0i)

        st2, y1 = pl.pallas_call(
            ph1,
            grid=(nsteps,),
            in_specs=[x_spec, cm_spec, full2((nbm, K)), full2((2, 64)),
                      full2((4, 4, 64)), full2((4, 64)), full2((1, 64)),
                      full2((4, 64, 64)), full2((1, 64))],
            out_specs=[full2((2, 64)), y1_spec],
            out_shape=[st_shape,
                       jax.ShapeDtypeStruct((ncell, NR, NT, 64),
                                            jnp.bfloat16)],
        )(xaug, cellmean, sumb, st1, w0f, w0s4, b0i, w1i, b1i)

        bout, sumb = pl.pallas_call(
            ph2,
            grid=(nsteps,),
            in_specs=[y1_spec, full2((nbm, K)), full2((2, 64)),
                      full2((4, 64)), full2((1, 1))],
            out_specs=[full2((nbm, K)), full2((nbm, K))],
            out_shape=[jax.ShapeDtypeStruct((nbm, K), f32),
                       jax.ShapeDtypeStruct((nbm, K), f32)],
            scratch_shapes=[pltpu.VMEM((nbm, K), f32)],
        )(y1, sumb, st2, w2i, b2i)
        bs.append(bout)

    bb = jnp.stack(bs, axis=0)
    bbh = jnp.transpose(bb, (1, 0, 2)).reshape(B, M, NRF, K)
    ah = jnp.transpose(A, (0, 2, 1, 3, 4, 5)).reshape(B, M, C, K, P)

    head = _make_head(M, K, P)
    hnew5, x5, zb = pl.pallas_call(
        head,
        grid=(B,),
        in_specs=[pl.BlockSpec((1, M, C, K, P), lambda b: (b, 0, 0, 0, 0)),
                  pl.BlockSpec((1, M, NRF, K), lambda b: (b, 0, 0, 0))],
        out_specs=[pl.BlockSpec((1, M, C, NRF, P), lambda b: (b, 0, 0, 0, 0)),
                   pl.BlockSpec((1, M, C, NRF, P), lambda b: (b, 0, 0, 0, 0)),
                   pl.BlockSpec((B, 1), lambda b: (0, 0))],
        out_shape=[jax.ShapeDtypeStruct((B, M, C, NRF, P), f32),
                   jax.ShapeDtypeStruct((B, M, C, NRF, P), f32),
                   jax.ShapeDtypeStruct((B, 1), f32)],
    )(ah, bbh)

    hnew = jnp.transpose(hnew5.reshape(B, M, C, NRF, NR, NT),
                         (0, 2, 1, 3, 4, 5))
    x_pred = jnp.transpose(x5.reshape(B, M, C, NRF, NR, NT),
                           (0, 2, 1, 3, 4, 5))
    y_pred = jnp.full((B, M), 1.0 / M, dtype=f32)
    z_pred = zb.reshape(B)
    return hnew, x_pred, y_pred, z_pred


# speedup vs baseline: 9.8578x; 1.0290x over previous
import jax
import jax.numpy as jnp
from jax import lax
from jax.experimental import pallas as pl
from jax.experimental.pallas import tpu as pltpu

_BN_EPS = 1e-5


def _make_prep(pcb, nr, nt):
    def body(at_ref, x_ref, cm_ref, gram_ref, scm_ref):
        s = pl.program_id(0)
        re = at_ref[0]
        im = at_ref[1]
        stre = jnp.sqrt(re * re + im * im + 1e-12)
        nrm = jnp.sqrt(jnp.sum(stre * stre, axis=1, keepdims=True) + 1e-12)
        x_ref[0] = re
        x_ref[1] = im
        x_ref[2] = stre
        x_ref[3] = stre / nrm
        x4 = x_ref[0:4]
        xr = x4.reshape(4, pcb, nr, nt)
        m_nr = jnp.mean(xr, axis=2)
        m_nt = jnp.mean(xr, axis=3)
        m_all = jnp.mean(m_nt, axis=2)
        cm_ref[...] = jnp.transpose(m_all)
        for c in range(4):
            x_ref[4 + c] = jnp.broadcast_to(
                m_nr[c][:, None, :], (pcb, nr, nt)).reshape(pcb, nr * nt)
            x_ref[8 + c] = jnp.broadcast_to(
                m_nt[c][:, :, None], (pcb, nr, nt)).reshape(pcb, nr * nt)
        cd = (((1,), (1,)), ((), ()))
        gx = lax.dot_general(x4.reshape(4, pcb * nr * nt),
                             x4.reshape(4, pcb * nr * nt), cd,
                             preferred_element_type=jnp.float32)
        gnr = lax.dot_general(m_nr.reshape(4, pcb * nt),
                              m_nr.reshape(4, pcb * nt), cd,
                              preferred_element_type=jnp.float32)
        gnt = lax.dot_general(m_nt.reshape(4, pcb * nr),
                              m_nt.reshape(4, pcb * nr), cd,
                              preferred_element_type=jnp.float32)
        gall = lax.dot_general(m_all, m_all, cd,
                               preferred_element_type=jnp.float32)
        gpart = jnp.stack([gx, gnr, gnt, gall])
        spart = jnp.sum(m_all, axis=1, keepdims=True)

        @pl.when(s == 0)
        def _():
            gram_ref[...] = gpart
            scm_ref[...] = spart

        @pl.when(s != 0)
        def _():
            gram_ref[...] = gram_ref[...] + gpart
            scm_ref[...] = scm_ref[...] + spart
    return body


def _make_bn0(npos, ncell, p):
    def body(sumb_ref, cm_ref, gram_ref, scm_ref, w0f_ref, w0s_ref,
             b0_ref, st_ref):
        sb = sumb_ref[...]
        cm = cm_ref[...]
        g = gram_ref[...]
        scm = scm_ref[...]
        w = w0f_ref[...]
        w0s = jnp.sum(w0s_ref[...], axis=0)
        b0 = b0_ref[0]
        ws = w[0] + w[1] + w[2] + w[3]

        ssum = jnp.sum(sb)
        ss2 = jnp.sum(sb * sb)
        svm = jnp.sum(jnp.sum(cm * sb[None], axis=2), axis=1,
                      keepdims=True)

        cdt = (((0,), (0,)), ((), ()))

        def dterm(wa, wb, gm):
            gwb = lax.dot_general(gm, wb, (((1,), (0,)), ((), ())),
                                  preferred_element_type=jnp.float32)
            return jnp.sum(wa * gwb, axis=0)

        scm_ws = lax.dot_general(scm, ws, cdt,
                                 preferred_element_type=jnp.float32)[0]
        svm_ws = lax.dot_general(svm, ws, cdt,
                                 preferred_element_type=jnp.float32)[0]

        mean_vec = p * scm_ws + npos * b0 + (p * ssum) * w0s
        sq = (dterm(w[0], w[0], g[0])
              + 4.0 * dterm(w[1], w[1], g[1])
              + 64.0 * dterm(w[2], w[2], g[2])
              + 256.0 * dterm(w[3], w[3], g[3])
              + 8.0 * dterm(w[0], w[1], g[1])
              + 128.0 * dterm(w[0], w[2], g[2])
              + 512.0 * (dterm(w[0], w[3], g[3])
                         + dterm(w[1], w[2], g[3])
                         + dterm(w[1], w[3], g[3])
                         + dterm(w[2], w[3], g[3]))
              + p * (ncell * b0 * b0 + 2.0 * ssum * b0 * w0s
                     + ss2 * w0s * w0s)
              + 2.0 * p * (b0 * scm_ws + w0s * svm_ws))
        st_ref[...] = jnp.stack([mean_vec, sq])
    return body


def _mu_rs(st, npos):
    mu = st[0] / npos
    var = st[1] / npos - mu * mu
    return mu, lax.rsqrt(var + _BN_EPS)


def _layer1_block(h, hb, w14, b1, cb, nr, nt):
    d = w14.shape[-1]
    m_nr = jnp.mean(h, axis=1)
    m_nt = jnp.mean(h, axis=2)
    m_all = jnp.mean(m_nt, axis=1)
    bf = jnp.bfloat16
    y = jnp.dot(hb.reshape(cb * nr * nt, d), w14[0],
                preferred_element_type=jnp.float32).reshape(cb, nr, nt, d)
    ynr = jnp.dot(m_nr.reshape(cb * nt, d).astype(bf), w14[1],
                  preferred_element_type=jnp.float32).reshape(cb, nt, d)
    ynt = jnp.dot(m_nt.reshape(cb * nr, d).astype(bf), w14[2],
                  preferred_element_type=jnp.float32).reshape(cb, nr, d)
    percell = jnp.dot(m_all.astype(bf), w14[3],
                      preferred_element_type=jnp.float32) + b1[None, :]
    y = y + ynr[:, None]
    y = y + (ynt + percell[:, None, :])[:, :, None, :]
    return y


def _make_ph1(cb, nr, nt, k, npos):
    def body(x_ref, cm_ref, sumb_ref, st1_ref, w0f_ref, w0s_ref, b0_ref,
             w1_ref, b1_ref, st_ref, y1_ref):
        s = pl.program_id(0)
        cbr = cb // k
        sumb_blk = sumb_ref[pl.ds(s * cbr, cbr), :].reshape(cb)
        mu1, rs1 = _mu_rs(st1_ref[...], npos)
        bf = jnp.bfloat16
        w = w0f_ref[...]
        wcat = (jnp.concatenate([w[0], w[1], w[2]], axis=0)
                * rs1[None, :]).astype(bf)
        w0s = jnp.sum(w0s_ref[...], axis=0) * rs1
        beff = (b0_ref[0] - mu1) * rs1
        cd = (((0,), (0,)), ((), ()))
        x12 = x_ref[...].astype(bf)
        y3 = lax.dot_general(x12.reshape(12, cb * nr * nt), wcat, cd,
                             preferred_element_type=jnp.float32
                             ).reshape(cb, nr * nt, 64)
        percell = lax.dot_general(cm_ref[...], w[3] * rs1[None, :],
                                  (((1,), (0,)), ((), ())),
                                  preferred_element_type=jnp.float32)
        ycorr = (percell[:, None, :] + beff[None, None, :]
                 + sumb_blk[:, None, None] * w0s[None, None, :])
        h0 = jnp.maximum(y3 + ycorr, 0.0).reshape(cb, nr, nt, 64)
        y1 = _layer1_block(h0, h0.astype(bf), w1_ref[...].astype(bf),
                           b1_ref[0], cb, nr, nt)
        d = y1.shape[-1]
        y1b = y1.astype(bf)
        y1_ref[...] = y1b
        y1f = y1b.reshape(cb * nr * nt, d)
        gram = lax.dot_general(y1f, y1f, (((0,), (0,)), ((), ())),
                               preferred_element_type=jnp.float32)
        eye = (lax.broadcasted_iota(jnp.int32, (d, d), 0)
               == lax.broadcasted_iota(jnp.int32, (d, d), 1))
        sq_vec = jnp.sum(jnp.where(eye, gram, 0.0), axis=1)
        ones = jnp.ones((1, cb * nr * nt), dtype=bf)
        sum_vec = lax.dot_general(ones, y1f, (((1,), (0,)), ((), ())),
                                  preferred_element_type=jnp.float32)[0]
        ps = jnp.stack([sum_vec, sq_vec])

        @pl.when(s == 0)
        def _():
            st_ref[...] = ps

        @pl.when(s != 0)
        def _():
            st_ref[...] = st_ref[...] + ps
    return body


def _make_ph2(cb, nr, nt, k, npos, nsteps):
    def body(y1_ref, sumb_ref, st2_ref, w2_ref, b2_ref,
             bout_ref, sumbn_ref, logit_ref):
        s = pl.program_id(0)
        cbr = cb // k
        mu2, rs2 = _mu_rs(st2_ref[...], npos)
        y1 = y1_ref[...].astype(jnp.float32)
        h1 = jnp.maximum(y1 * rs2[None, None, None, :]
                         - (mu2 * rs2)[None, None, None, :], 0.0)
        mbar = jnp.mean(h1, axis=(1, 2))
        w2s = jnp.sum(w2_ref[...], axis=0)
        ylog = jnp.dot(mbar, w2s[:, None],
                       preferred_element_type=jnp.float32)
        ylog = ylog + b2_ref[0, 0]
        logit_ref[pl.ds(s * cbr, cbr), :] = ylog.reshape(cbr, k)

        @pl.when(s == nsteps - 1)
        def _():
            logits = logit_ref[...]
            mx = jnp.max(logits, axis=1, keepdims=True)
            e = jnp.exp(logits - mx)
            p = e / jnp.sum(e, axis=1, keepdims=True)
            bout_ref[...] = p
            sumbn_ref[...] = sumb_ref[...] + p
    return body


def _make_head(m, k, p):
    def body(a_ref, bb_ref, hnew_ref, x_ref, z_ref):
        ablk = a_ref[0]
        bblk = bb_ref[0]
        hn = lax.dot_general(bblk, ablk, (((2,), (2,)), ((0,), (0,))),
                             preferred_element_type=jnp.float32)
        g = jnp.sum(hn * hn, axis=(2, 3))
        hnt = jnp.transpose(hn, (0, 2, 1, 3))
        hnew_ref[0] = hnt
        x_ref[0] = hnt / (jnp.sqrt(g)[:, None, :, None] + 1e-8)
        z_ref[pl.ds(pl.program_id(0), 1), :] = (
            jnp.sum(jnp.log2(1.0 + g)).reshape(1, 1))
    return body


def kernel(A, params):
    B, C, M, K, NR, NT = A.shape
    NRF = params["W0"].shape[0]
    P = NR * NT
    ncell = B * M * K
    nbm = B * M
    npos = float(ncell * P)
    cb = min(128, ncell)
    nsteps = ncell // cb
    f32 = jnp.float32

    at = jnp.transpose(A, (1, 0, 2, 3, 4, 5)).reshape(C, ncell, P)
    pcb = min(128, ncell)
    prep = _make_prep(pcb, NR, NT)
    xaug, cellmean, grams, scm = pl.pallas_call(
        prep,
        grid=(ncell // pcb,),
        in_specs=[pl.BlockSpec((C, pcb, P), lambda s: (0, s, 0))],
        out_specs=[pl.BlockSpec((12, pcb, P), lambda s: (0, s, 0)),
                   pl.BlockSpec((pcb, 4), lambda s: (s, 0)),
                   pl.BlockSpec((4, 4, 4), lambda s: (0, 0, 0)),
                   pl.BlockSpec((4, 1), lambda s: (0, 0))],
        out_shape=[jax.ShapeDtypeStruct((12, ncell, P), f32),
                   jax.ShapeDtypeStruct((ncell, 4), f32),
                   jax.ShapeDtypeStruct((4, 4, 4), f32),
                   jax.ShapeDtypeStruct((4, 1), f32)],
    )(at)
    cm3 = jnp.transpose(cellmean).reshape(4, nbm, K)

    full2 = lambda shape: pl.BlockSpec(shape, lambda s: tuple(0 for _ in shape))
    x_spec = pl.BlockSpec((12, cb, P), lambda s: (0, s, 0))
    cm_spec = pl.BlockSpec((cb, 4), lambda s: (s, 0))
    y1_spec = pl.BlockSpec((cb, NR, NT, 64), lambda s: (s, 0, 0, 0))
    st_shape = jax.ShapeDtypeStruct((2, 64), f32)

    bn0 = _make_bn0(npos, float(ncell), float(P))
    ph1 = _make_ph1(cb, NR, NT, K, npos)
    ph2 = _make_ph2(cb, NR, NT, K, npos, nsteps)

    W0, W1, W2 = params["W0"], params["W1"], params["W2"]
    b0a, b1a, b2a = params["b0"], params["b1"], params["b2"]

    sumb = jnp.zeros((nbm, K), f32)
    bs = []
    for i in range(NRF):
        w0f = W0[i, :, :4, :]
        w0s4 = W0[i, :, 4, :]
        b0i = b0a[i:i + 1]
        w1i = W1[i]
        b1i = b1a[i:i + 1]
        w2i = W2[i, :, :, 0]
        b2i = b2a[i:i + 1]

        st1 = pl.pallas_call(
            bn0,
            grid=(1,),
            in_specs=[full2((nbm, K)), full2((4, nbm, K)), full2((4, 4, 4)),
                      full2((4, 1)), full2((4, 4, 64)), full2((4, 64)),
                      full2((1, 64))],
            out_specs=full2((2, 64)),
            out_shape=st_shape,
        )(sumb, cm3, grams, scm, w0f, w0s4, b0i)

        st2, y1 = pl.pallas_call(
            ph1,
            grid=(nsteps,),
            in_specs=[x_spec, cm_spec, full2((nbm, K)), full2((2, 64)),
                      full2((4, 4, 64)), full2((4, 64)), full2((1, 64)),
                      full2((4, 64, 64)), full2((1, 64))],
            out_specs=[full2((2, 64)), y1_spec],
            out_shape=[st_shape,
                       jax.ShapeDtypeStruct((ncell, NR, NT, 64),
                                            jnp.bfloat16)],
        )(xaug, cellmean, sumb, st1, w0f, w0s4, b0i, w1i, b1i)

        bout, sumb = pl.pallas_call(
            ph2,
            grid=(nsteps,),
            in_specs=[y1_spec, full2((nbm, K)), full2((2, 64)),
                      full2((4, 64)), full2((1, 1))],
            out_specs=[full2((nbm, K)), full2((nbm, K))],
            out_shape=[jax.ShapeDtypeStruct((nbm, K), f32),
                       jax.ShapeDtypeStruct((nbm, K), f32)],
            scratch_shapes=[pltpu.VMEM((nbm, K), f32)],
        )(y1, sumb, st2, w2i, b2i)
        bs.append(bout)

    bb = jnp.stack(bs, axis=0)
    bbh = jnp.transpose(bb, (1, 0, 2)).reshape(B, M, NRF, K)
    ah = jnp.transpose(A, (0, 2, 1, 3, 4, 5)).reshape(B, M, C, K, P)

    head = _make_head(M, K, P)
    hnew5, x5, zb = pl.pallas_call(
        head,
        grid=(B,),
        in_specs=[pl.BlockSpec((1, M, C, K, P), lambda b: (b, 0, 0, 0, 0)),
                  pl.BlockSpec((1, M, NRF, K), lambda b: (b, 0, 0, 0))],
        out_specs=[pl.BlockSpec((1, M, C, NRF, P), lambda b: (b, 0, 0, 0, 0)),
                   pl.BlockSpec((1, M, C, NRF, P), lambda b: (b, 0, 0, 0, 0)),
                   pl.BlockSpec((B, 1), lambda b: (0, 0))],
        out_shape=[jax.ShapeDtypeStruct((B, M, C, NRF, P), f32),
                   jax.ShapeDtypeStruct((B, M, C, NRF, P), f32),
                   jax.ShapeDtypeStruct((B, 1), f32)],
    )(ah, bbh)

    hnew = jnp.transpose(hnew5.reshape(B, M, C, NRF, NR, NT),
                         (0, 2, 1, 3, 4, 5))
    x_pred = jnp.transpose(x5.reshape(B, M, C, NRF, NR, NT),
                           (0, 2, 1, 3, 4, 5))
    y_pred = jnp.full((B, M), 1.0 / M, dtype=f32)
    z_pred = zb.reshape(B)
    return hnew, x_pred, y_pred, z_pred
